# Initial kernel scaffold; baseline (speedup 1.0000x reference)
import jax
import jax.numpy as jnp
from jax.experimental import pallas as pl


def kernel(x, edge_index, edge_attr, nnW1, nnb1, nng1, nnbt1, root1, bias1, nnW2, nnb2, nng2, nnbt2, root2, bias2, lin1W, lin1b, lin1g, lin1bt, m1W, m1b, m1g, m1bt, m2W, m2b, m2g, m2bt):
    raise NotImplementedError("write your pallas kernel here")


# speedup vs baseline: 2.8538x; 2.8538x over previous
import functools

import jax
import jax.numpy as jnp
from jax import lax
from jax.experimental import pallas as pl
from jax.experimental.pallas import tpu as pltpu
from jax.experimental.pallas import tpu_sc as plsc

N = 10000
NE = 160000
D = 16
E_DIM = 4
Q = 96
C = 8
EPS = 1e-5

NC = 2
NS = 16
NW = NC * NS
EPW = NE // NW
WIN = 125
NCHUNK = EPW // WIN
FIRE = 8
NPS = N // NS

ET = 2000
NT = NE // ET
DD = D * D

@functools.cache
def _sc_mesh():
    return plsc.VectorSubcoreMesh(core_axis_name="c", subcore_axis_name="s",
                                  num_cores=NC, num_subcores=NS)


def _dot_bf16(a, b):
    return jnp.dot(a.astype(jnp.bfloat16), b.astype(jnp.bfloat16),
                   preferred_element_type=jnp.float32)


def _gather_body(x_hbm, src_hbm, out_hbm, idx_v, rows_v, sem):
    cid = lax.axis_index("c")
    sid = lax.axis_index("s")
    wid = sid * NC + cid
    pltpu.sync_copy(src_hbm.at[wid], idx_v)

    @pl.loop(0, NCHUNK, step=FIRE)
    def _(j0):
        cps = [
            pltpu.async_copy(
                x_hbm.at[idx_v.at[j0 + k]],
                rows_v.at[pl.ds((j0 + k) * WIN, WIN)],
                sem,
            )
            for k in range(FIRE)
        ]
        for cp in cps:
            cp.wait()

    pltpu.sync_copy(rows_v, out_hbm.at[pl.ds(wid * EPW, EPW)])


@jax.jit
def _sc_gather(x, src3):
    return pl.kernel(
        _gather_body,
        out_type=jax.ShapeDtypeStruct((NE, D), jnp.float32),
        mesh=_sc_mesh(),
        compiler_params=pltpu.CompilerParams(use_tc_tiling_on_sc=False),
        scratch_types=[
            pltpu.VMEM((NCHUNK, WIN), jnp.int32),
            pltpu.VMEM((EPW, D), jnp.float32),
            pltpu.SemaphoreType.DMA,
        ],
    )(x, src3)


def _scatter_body(msg_hbm, dst_hbm, zero_hbm, out_hbm, idx_v, vals_v, acc_sh, sem):
    cid = lax.axis_index("c")
    sid = lax.axis_index("s")
    wid = sid * NC + cid
    pltpu.sync_copy(zero_hbm.at[pl.ds(sid * NPS, NPS)],
                    acc_sh.at[pl.ds(sid * NPS, NPS)])
    pltpu.sync_copy(dst_hbm.at[wid], idx_v)
    pltpu.sync_copy(msg_hbm.at[pl.ds(wid * EPW, EPW)], vals_v)
    plsc.subcore_barrier()

    @pl.loop(0, NCHUNK, step=FIRE)
    def _(j0):
        cps = [
            pltpu.async_copy(
                vals_v.at[pl.ds((j0 + k) * WIN, WIN)],
                acc_sh.at[idx_v.at[j0 + k]],
                sem,
                add=True,
            )
            for k in range(FIRE)
        ]
        for cp in cps:
            cp.wait()

    plsc.subcore_barrier()
    pltpu.sync_copy(acc_sh.at[pl.ds(sid * NPS, NPS)],
                    out_hbm.at[cid, pl.ds(sid * NPS, NPS)])


@jax.jit
def _sc_scatter(msg, dst3, zeros_nd):
    return pl.kernel(
        _scatter_body,
        out_type=jax.ShapeDtypeStruct((NC, N, D), jnp.float32),
        mesh=_sc_mesh(),
        compiler_params=pltpu.CompilerParams(use_tc_tiling_on_sc=False),
        scratch_types=[
            pltpu.VMEM((NCHUNK, WIN), jnp.int32),
            pltpu.VMEM((EPW, D), jnp.float32),
            pltpu.VMEM_SHARED((N, D), jnp.float32),
            pltpu.SemaphoreType.DMA,
        ],
    )(msg, dst3, zeros_nd)


def _stats_kernel(ea_ref, w1_ref, b1_ref, w2_ref, b2_ref,
                  s1_ref, ss1_ref, s2_ref, ss2_ref):
    i = pl.program_id(0)

    @pl.when(i == 0)
    def _():
        s1_ref[...] = jnp.zeros_like(s1_ref)
        ss1_ref[...] = jnp.zeros_like(ss1_ref)
        s2_ref[...] = jnp.zeros_like(s2_ref)
        ss2_ref[...] = jnp.zeros_like(ss2_ref)

    ea = ea_ref[...]
    r1 = jnp.maximum(_dot_bf16(ea, w1_ref[...]) + b1_ref[...], 0.0)
    r2 = jnp.maximum(_dot_bf16(ea, w2_ref[...]) + b2_ref[...], 0.0)
    s1_ref[...] += jnp.sum(r1, axis=0, keepdims=True)
    ss1_ref[...] += jnp.sum(r1 * r1, axis=0, keepdims=True)
    s2_ref[...] += jnp.sum(r2, axis=0, keepdims=True)
    ss2_ref[...] += jnp.sum(r2 * r2, axis=0, keepdims=True)


@jax.jit
def _tc_stats(edge_attr, w1, b1, w2, b2):
    one_row = pl.BlockSpec((1, DD), lambda i: (0, 0))
    return pl.pallas_call(
        _stats_kernel,
        grid=(NT,),
        in_specs=[
            pl.BlockSpec((ET, E_DIM), lambda i: (i, 0)),
            pl.BlockSpec((E_DIM, DD), lambda i: (0, 0)),
            pl.BlockSpec((1, DD), lambda i: (0, 0)),
            pl.BlockSpec((E_DIM, DD), lambda i: (0, 0)),
            pl.BlockSpec((1, DD), lambda i: (0, 0)),
        ],
        out_specs=[one_row, one_row, one_row, one_row],
        out_shape=[jax.ShapeDtypeStruct((1, DD), jnp.float32)] * 4,
    )(edge_attr, w1, b1, w2, b2)


def _msg_kernel(ea_ref, xj_ref, w_ref, b_ref, g_ref, bt_ref,
                s_ref, ss_ref, k1_ref, k2_ref, msg_ref):
    mu = s_ref[...] * (1.0 / NE)
    var = ss_ref[...] * (1.0 / NE) - mu * mu
    inv = jax.lax.rsqrt(var + EPS)
    sc = g_ref[...] * inv
    sh = bt_ref[...] - mu * sc

    ea = ea_ref[...]
    xj = xj_ref[...]
    a = _dot_bf16(ea, w_ref[...]) + b_ref[...]
    we = sc * jnp.maximum(a, 0.0) + sh
    xjb = _dot_bf16(xj, k1_ref[...])
    web = we.astype(jnp.bfloat16).astype(jnp.float32)
    msg_ref[...] = jnp.dot(xjb * web, k2_ref[...],
                           preferred_element_type=jnp.float32,
                           precision=lax.Precision.HIGHEST)


@jax.jit
def _tc_msg(edge_attr, xj, w, b, g, bt, s, ss, k1, k2):
    row = pl.BlockSpec((1, DD), lambda i: (0, 0))
    return pl.pallas_call(
        _msg_kernel,
        grid=(NT,),
        in_specs=[
            pl.BlockSpec((ET, E_DIM), lambda i: (i, 0)),
            pl.BlockSpec((ET, D), lambda i: (i, 0)),
            pl.BlockSpec((E_DIM, DD), lambda i: (0, 0)),
            row, row, row, row, row,
            pl.BlockSpec((D, DD), lambda i: (0, 0)),
            pl.BlockSpec((DD, D), lambda i: (0, 0)),
        ],
        out_specs=pl.BlockSpec((ET, D), lambda i: (i, 0)),
        out_shape=jax.ShapeDtypeStruct((NE, D), jnp.float32),
    )(edge_attr, xj, w, b, g, bt, s, ss, k1, k2)


def _combine_kernel(p_ref, x_ref, root_ref, bias_ref, out_ref):
    out_ref[...] = (
        p_ref[0] + p_ref[1]
        + _dot_bf16(x_ref[...], root_ref[...])
        + bias_ref[...]
    )


@jax.jit
def _tc_combine(partials, x, root, bias_row):
    return pl.pallas_call(
        _combine_kernel,
        out_shape=jax.ShapeDtypeStruct((N, D), jnp.float32),
    )(partials, x, root, bias_row)


def _bn_rows(h, g, bt):
    mu = jnp.mean(h, axis=0, keepdims=True)
    dv = h - mu
    var = jnp.mean(dv * dv, axis=0, keepdims=True)
    return g * dv * jax.lax.rsqrt(var + EPS) + bt


def _head_kernel(x1_ref, p_ref, root2_ref, bias2_ref,
                 l1a_ref, l1b_ref, l1bias_ref, l1g_ref, l1bt_ref,
                 m1w_ref, m1b_ref, m1g_ref, m1bt_ref,
                 m2w_ref, m2b_ref, m2g_ref, m2bt_ref, out_ref):
    x1 = x1_ref[...]
    x2 = (p_ref[0] + p_ref[1]
          + _dot_bf16(x1, root2_ref[...])
          + bias2_ref[...])
    h = (_dot_bf16(x1, l1a_ref[...]) + _dot_bf16(x2, l1b_ref[...])
         + l1bias_ref[...])
    h = _bn_rows(jnp.maximum(h, 0.0), l1g_ref[...], l1bt_ref[...])
    h = _dot_bf16(h, m1w_ref[...]) + m1b_ref[...]
    h = _bn_rows(jnp.maximum(h, 0.0), m1g_ref[...], m1bt_ref[...])
    h = _dot_bf16(h, m2w_ref[...]) + m2b_ref[...]
    out_ref[...] = _bn_rows(jnp.maximum(h, 0.0), m2g_ref[...], m2bt_ref[...])


@jax.jit
def _tc_head(x1, partials2, root2, bias2_row, l1a, l1b, l1bias, l1g, l1bt,
             m1w, m1b, m1g, m1bt, m2w, m2b, m2g, m2bt):
    return pl.pallas_call(
        _head_kernel,
        out_shape=jax.ShapeDtypeStruct((N, C), jnp.float32),
    )(x1, partials2, root2, bias2_row, l1a, l1b, l1bias, l1g, l1bt,
      m1w, m1b, m1g, m1bt, m2w, m2b, m2g, m2bt)


def kernel(x, edge_index, edge_attr, nnW1, nnb1, nng1, nnbt1, root1, bias1,
           nnW2, nnb2, nng2, nnbt2, root2, bias2, lin1W, lin1b, lin1g, lin1bt,
           m1W, m1b, m1g, m1bt, m2W, m2b, m2g, m2bt):
    f32 = jnp.float32
    src3 = edge_index[0].astype(jnp.int32).reshape(NW, NCHUNK, WIN)
    dst3 = edge_index[1].astype(jnp.int32).reshape(NW, NCHUNK, WIN)
    zeros_nd = jnp.zeros((N, D), f32)

    b1r, g1r, bt1r = (v.reshape(1, DD) for v in (nnb1, nng1, nnbt1))
    b2r, g2r, bt2r = (v.reshape(1, DD) for v in (nnb2, nng2, nnbt2))
    bias1r = bias1.reshape(1, D)
    bias2r = bias2.reshape(1, D)
    l1biasr, l1gr, l1btr = (v.reshape(1, Q) for v in (lin1b, lin1g, lin1bt))
    m1br, m1gr, m1btr = (v.reshape(1, Q) for v in (m1b, m1g, m1bt))
    m2br, m2gr, m2btr = (v.reshape(1, C) for v in (m2b, m2g, m2bt))
    l1a = lin1W[:D]
    l1b = lin1W[D:]

    eye = jnp.eye(D, dtype=f32)
    k1 = jnp.kron(eye, jnp.ones((1, D), f32))
    k2 = jnp.kron(jnp.ones((D, 1), f32), eye)

    s1, ss1, s2, ss2 = _tc_stats(edge_attr, nnW1, b1r, nnW2, b2r)

    xj1 = _sc_gather(x, src3)
    msg1 = _tc_msg(edge_attr, xj1, nnW1, b1r, g1r, bt1r, s1, ss1, k1, k2)
    p1 = _sc_scatter(msg1, dst3, zeros_nd)
    x1 = _tc_combine(p1, x, root1, bias1r)

    xj2 = _sc_gather(x1, src3)
    msg2 = _tc_msg(edge_attr, xj2, nnW2, b2r, g2r, bt2r, s2, ss2, k1, k2)
    p2 = _sc_scatter(msg2, dst3, zeros_nd)

    return _tc_head(x1, p2, root2, bias2r, l1a, l1b, l1biasr, l1gr, l1btr,
                    m1W, m1br, m1gr, m1btr, m2W, m2br, m2gr, m2btr)


# speedup vs baseline: 3.6748x; 1.2877x over previous
import functools

import jax
import jax.numpy as jnp
from jax import lax
from jax.experimental import pallas as pl
from jax.experimental.pallas import tpu as pltpu
from jax.experimental.pallas import tpu_sc as plsc

N = 10000
NE = 160000
D = 16
E_DIM = 4
Q = 96
C = 8
EPS = 1e-5

NC = 2
NS = 16
NW = NC * NS
EPW = NE // NW
WIN = 125
NCHUNK = EPW // WIN
FIRE = 8
NPS = N // NS

ET = 4000
NT = NE // ET
DD = D * D

@functools.cache
def _sc_mesh():
    return plsc.VectorSubcoreMesh(core_axis_name="c", subcore_axis_name="s",
                                  num_cores=NC, num_subcores=NS)


def _dot_bf16(a, b):
    return jnp.dot(a.astype(jnp.bfloat16), b.astype(jnp.bfloat16),
                   preferred_element_type=jnp.float32)


def _gather_body(x_hbm, src_hbm, out_hbm, idx_v, rows_v, sem):
    cid = lax.axis_index("c")
    sid = lax.axis_index("s")
    wid = sid * NC + cid
    pltpu.sync_copy(src_hbm.at[wid], idx_v)

    @pl.loop(0, NCHUNK, step=FIRE)
    def _(j0):
        cps = [
            pltpu.async_copy(
                x_hbm.at[idx_v.at[j0 + k]],
                rows_v.at[pl.ds((j0 + k) * WIN, WIN)],
                sem,
            )
            for k in range(FIRE)
        ]
        for cp in cps:
            cp.wait()

    pltpu.sync_copy(rows_v, out_hbm.at[pl.ds(wid * EPW, EPW)])


@jax.jit
def _sc_gather(x, src3):
    return pl.kernel(
        _gather_body,
        out_type=jax.ShapeDtypeStruct((NE, D), jnp.float32),
        mesh=_sc_mesh(),
        compiler_params=pltpu.CompilerParams(use_tc_tiling_on_sc=False),
        scratch_types=[
            pltpu.VMEM((NCHUNK, WIN), jnp.int32),
            pltpu.VMEM((EPW, D), jnp.float32),
            pltpu.SemaphoreType.DMA,
        ],
    )(x, src3)


def _scatter_body(msg_hbm, dst_hbm, zero_hbm, out_hbm, idx_v, vals_v, acc_sh, sem):
    cid = lax.axis_index("c")
    sid = lax.axis_index("s")
    wid = sid * NC + cid
    pltpu.sync_copy(zero_hbm.at[pl.ds(sid * NPS, NPS)],
                    acc_sh.at[pl.ds(sid * NPS, NPS)])
    pltpu.sync_copy(dst_hbm.at[wid], idx_v)
    pltpu.sync_copy(msg_hbm.at[pl.ds(wid * EPW, EPW)], vals_v)
    plsc.subcore_barrier()

    @pl.loop(0, NCHUNK, step=FIRE)
    def _(j0):
        cps = [
            pltpu.async_copy(
                vals_v.at[pl.ds((j0 + k) * WIN, WIN)],
                acc_sh.at[idx_v.at[j0 + k]],
                sem,
                add=True,
            )
            for k in range(FIRE)
        ]
        for cp in cps:
            cp.wait()

    plsc.subcore_barrier()
    pltpu.sync_copy(acc_sh.at[pl.ds(sid * NPS, NPS)],
                    out_hbm.at[cid, pl.ds(sid * NPS, NPS)])


@jax.jit
def _sc_scatter(msg, dst3, zeros_nd):
    return pl.kernel(
        _scatter_body,
        out_type=jax.ShapeDtypeStruct((NC, N, D), jnp.float32),
        mesh=_sc_mesh(),
        compiler_params=pltpu.CompilerParams(use_tc_tiling_on_sc=False),
        scratch_types=[
            pltpu.VMEM((NCHUNK, WIN), jnp.int32),
            pltpu.VMEM((EPW, D), jnp.float32),
            pltpu.VMEM_SHARED((N, D), jnp.float32),
            pltpu.SemaphoreType.DMA,
        ],
    )(msg, dst3, zeros_nd)


def _stats_kernel(ea_ref, w1_ref, b1_ref, w2_ref, b2_ref,
                  s1_ref, ss1_ref, s2_ref, ss2_ref):
    i = pl.program_id(0)

    @pl.when(i == 0)
    def _():
        s1_ref[...] = jnp.zeros_like(s1_ref)
        ss1_ref[...] = jnp.zeros_like(ss1_ref)
        s2_ref[...] = jnp.zeros_like(s2_ref)
        ss2_ref[...] = jnp.zeros_like(ss2_ref)

    ea = ea_ref[...]
    r1 = jnp.maximum(_dot_bf16(ea, w1_ref[...]) + b1_ref[...], 0.0)
    r2 = jnp.maximum(_dot_bf16(ea, w2_ref[...]) + b2_ref[...], 0.0)
    s1_ref[...] += jnp.sum(r1, axis=0, keepdims=True)
    ss1_ref[...] += jnp.sum(r1 * r1, axis=0, keepdims=True)
    s2_ref[...] += jnp.sum(r2, axis=0, keepdims=True)
    ss2_ref[...] += jnp.sum(r2 * r2, axis=0, keepdims=True)


@jax.jit
def _tc_stats(edge_attr, w1, b1, w2, b2):
    one_row = pl.BlockSpec((1, DD), lambda i: (0, 0))
    return pl.pallas_call(
        _stats_kernel,
        grid=(NT,),
        in_specs=[
            pl.BlockSpec((ET, E_DIM), lambda i: (i, 0)),
            pl.BlockSpec((E_DIM, DD), lambda i: (0, 0)),
            pl.BlockSpec((1, DD), lambda i: (0, 0)),
            pl.BlockSpec((E_DIM, DD), lambda i: (0, 0)),
            pl.BlockSpec((1, DD), lambda i: (0, 0)),
        ],
        out_specs=[one_row, one_row, one_row, one_row],
        out_shape=[jax.ShapeDtypeStruct((1, DD), jnp.float32)] * 4,
    )(edge_attr, w1, b1, w2, b2)


def _msg_kernel(ea_ref, xj_ref, w_ref, b_ref, g_ref, bt_ref,
                s_ref, ss_ref, k1_ref, k2_ref, msg_ref):
    mu = s_ref[...] * (1.0 / NE)
    var = ss_ref[...] * (1.0 / NE) - mu * mu
    inv = jax.lax.rsqrt(var + EPS)
    sc = g_ref[...] * inv
    sh = bt_ref[...] - mu * sc

    ea = ea_ref[...]
    xj = xj_ref[...]
    a = _dot_bf16(ea, w_ref[...]) + b_ref[...]
    we = sc * jnp.maximum(a, 0.0) + sh
    xjb = _dot_bf16(xj, k1_ref[...])
    web = we.astype(jnp.bfloat16).astype(jnp.float32)
    p = xjb * web
    p_hi = p.astype(jnp.bfloat16).astype(jnp.float32)
    p_lo = p - p_hi
    msg_ref[...] = _dot_bf16(p_hi, k2_ref[...]) + _dot_bf16(p_lo, k2_ref[...])


@jax.jit
def _tc_msg(edge_attr, xj, w, b, g, bt, s, ss, k1, k2):
    row = pl.BlockSpec((1, DD), lambda i: (0, 0))
    return pl.pallas_call(
        _msg_kernel,
        grid=(NT,),
        in_specs=[
            pl.BlockSpec((ET, E_DIM), lambda i: (i, 0)),
            pl.BlockSpec((ET, D), lambda i: (i, 0)),
            pl.BlockSpec((E_DIM, DD), lambda i: (0, 0)),
            row, row, row, row, row,
            pl.BlockSpec((D, DD), lambda i: (0, 0)),
            pl.BlockSpec((DD, D), lambda i: (0, 0)),
        ],
        out_specs=pl.BlockSpec((ET, D), lambda i: (i, 0)),
        out_shape=jax.ShapeDtypeStruct((NE, D), jnp.float32),
    )(edge_attr, xj, w, b, g, bt, s, ss, k1, k2)


def _combine_kernel(p_ref, x_ref, root_ref, bias_ref, out_ref):
    out_ref[...] = (
        p_ref[0] + p_ref[1]
        + _dot_bf16(x_ref[...], root_ref[...])
        + bias_ref[...]
    )


@jax.jit
def _tc_combine(partials, x, root, bias_row):
    return pl.pallas_call(
        _combine_kernel,
        out_shape=jax.ShapeDtypeStruct((N, D), jnp.float32),
    )(partials, x, root, bias_row)


def _bn_rows(h, g, bt):
    mu = jnp.mean(h, axis=0, keepdims=True)
    dv = h - mu
    var = jnp.mean(dv * dv, axis=0, keepdims=True)
    return g * dv * jax.lax.rsqrt(var + EPS) + bt


def _head_kernel(x1_ref, p_ref, root2_ref, bias2_ref,
                 l1a_ref, l1b_ref, l1bias_ref, l1g_ref, l1bt_ref,
                 m1w_ref, m1b_ref, m1g_ref, m1bt_ref,
                 m2w_ref, m2b_ref, m2g_ref, m2bt_ref, out_ref):
    x1 = x1_ref[...]
    x2 = (p_ref[0] + p_ref[1]
          + _dot_bf16(x1, root2_ref[...])
          + bias2_ref[...])
    h = (_dot_bf16(x1, l1a_ref[...]) + _dot_bf16(x2, l1b_ref[...])
         + l1bias_ref[...])
    h = _bn_rows(jnp.maximum(h, 0.0), l1g_ref[...], l1bt_ref[...])
    h = _dot_bf16(h, m1w_ref[...]) + m1b_ref[...]
    h = _bn_rows(jnp.maximum(h, 0.0), m1g_ref[...], m1bt_ref[...])
    h = _dot_bf16(h, m2w_ref[...]) + m2b_ref[...]
    out_ref[...] = _bn_rows(jnp.maximum(h, 0.0), m2g_ref[...], m2bt_ref[...])


@jax.jit
def _tc_head(x1, partials2, root2, bias2_row, l1a, l1b, l1bias, l1g, l1bt,
             m1w, m1b, m1g, m1bt, m2w, m2b, m2g, m2bt):
    return pl.pallas_call(
        _head_kernel,
        out_shape=jax.ShapeDtypeStruct((N, C), jnp.float32),
    )(x1, partials2, root2, bias2_row, l1a, l1b, l1bias, l1g, l1bt,
      m1w, m1b, m1g, m1bt, m2w, m2b, m2g, m2bt)


def kernel(x, edge_index, edge_attr, nnW1, nnb1, nng1, nnbt1, root1, bias1,
           nnW2, nnb2, nng2, nnbt2, root2, bias2, lin1W, lin1b, lin1g, lin1bt,
           m1W, m1b, m1g, m1bt, m2W, m2b, m2g, m2bt):
    f32 = jnp.float32
    src3 = edge_index[0].astype(jnp.int32).reshape(NW, NCHUNK, WIN)
    dst3 = edge_index[1].astype(jnp.int32).reshape(NW, NCHUNK, WIN)
    zeros_nd = jnp.zeros((N, D), f32)

    b1r, g1r, bt1r = (v.reshape(1, DD) for v in (nnb1, nng1, nnbt1))
    b2r, g2r, bt2r = (v.reshape(1, DD) for v in (nnb2, nng2, nnbt2))
    bias1r = bias1.reshape(1, D)
    bias2r = bias2.reshape(1, D)
    l1biasr, l1gr, l1btr = (v.reshape(1, Q) for v in (lin1b, lin1g, lin1bt))
    m1br, m1gr, m1btr = (v.reshape(1, Q) for v in (m1b, m1g, m1bt))
    m2br, m2gr, m2btr = (v.reshape(1, C) for v in (m2b, m2g, m2bt))
    l1a = lin1W[:D]
    l1b = lin1W[D:]

    eye = jnp.eye(D, dtype=f32)
    k1 = jnp.kron(eye, jnp.ones((1, D), f32))
    k2 = jnp.kron(jnp.ones((D, 1), f32), eye)

    s1, ss1, s2, ss2 = _tc_stats(edge_attr, nnW1, b1r, nnW2, b2r)

    xj1 = _sc_gather(x, src3)
    msg1 = _tc_msg(edge_attr, xj1, nnW1, b1r, g1r, bt1r, s1, ss1, k1, k2)
    p1 = _sc_scatter(msg1, dst3, zeros_nd)
    x1 = _tc_combine(p1, x, root1, bias1r)

    xj2 = _sc_gather(x1, src3)
    msg2 = _tc_msg(edge_attr, xj2, nnW2, b2r, g2r, bt2r, s2, ss2, k1, k2)
    p2 = _sc_scatter(msg2, dst3, zeros_nd)

    return _tc_head(x1, p2, root2, bias2r, l1a, l1b, l1biasr, l1gr, l1btr,
                    m1W, m1br, m1gr, m1btr, m2W, m2br, m2gr, m2btr)


# speedup vs baseline: 3.7575x; 1.0225x over previous
import functools

import jax
import jax.numpy as jnp
from jax import lax
from jax.experimental import pallas as pl
from jax.experimental.pallas import tpu as pltpu
from jax.experimental.pallas import tpu_sc as plsc

N = 10000
NE = 160000
D = 16
E_DIM = 4
Q = 96
C = 8
EPS = 1e-5

NC = 2
NS = 16
NW = NC * NS
EPW = NE // NW
WIN = 125
NCHUNK = EPW // WIN
FIRE = 8
NPS = N // NS

ET = 8000
NT = NE // ET
DD = D * D

@functools.cache
def _sc_mesh():
    return plsc.VectorSubcoreMesh(core_axis_name="c", subcore_axis_name="s",
                                  num_cores=NC, num_subcores=NS)


def _dot_bf16(a, b):
    return jnp.dot(a.astype(jnp.bfloat16), b.astype(jnp.bfloat16),
                   preferred_element_type=jnp.float32)


def _gather_body(x_hbm, src_hbm, out_hbm, idx_v, rows_v, sem):
    cid = lax.axis_index("c")
    sid = lax.axis_index("s")
    wid = sid * NC + cid
    pltpu.sync_copy(src_hbm.at[wid], idx_v)

    @pl.loop(0, NCHUNK, step=FIRE)
    def _(j0):
        cps = [
            pltpu.async_copy(
                x_hbm.at[idx_v.at[j0 + k]],
                rows_v.at[pl.ds((j0 + k) * WIN, WIN)],
                sem,
            )
            for k in range(FIRE)
        ]
        for cp in cps:
            cp.wait()

    pltpu.sync_copy(rows_v, out_hbm.at[pl.ds(wid * EPW, EPW)])


@jax.jit
def _sc_gather(x, src3):
    return pl.kernel(
        _gather_body,
        out_type=jax.ShapeDtypeStruct((NE, D), jnp.float32),
        mesh=_sc_mesh(),
        compiler_params=pltpu.CompilerParams(use_tc_tiling_on_sc=False),
        scratch_types=[
            pltpu.VMEM((NCHUNK, WIN), jnp.int32),
            pltpu.VMEM((EPW, D), jnp.float32),
            pltpu.SemaphoreType.DMA,
        ],
    )(x, src3)


def _scatter_body(msg_hbm, dst_hbm, zero_hbm, out_hbm, idx_v, vals_v, acc_sh, sem):
    cid = lax.axis_index("c")
    sid = lax.axis_index("s")
    wid = sid * NC + cid
    pltpu.sync_copy(zero_hbm.at[pl.ds(sid * NPS, NPS)],
                    acc_sh.at[pl.ds(sid * NPS, NPS)])
    pltpu.sync_copy(dst_hbm.at[wid], idx_v)
    pltpu.sync_copy(msg_hbm.at[pl.ds(wid * EPW, EPW)], vals_v)
    plsc.subcore_barrier()

    @pl.loop(0, NCHUNK, step=FIRE)
    def _(j0):
        cps = [
            pltpu.async_copy(
                vals_v.at[pl.ds((j0 + k) * WIN, WIN)],
                acc_sh.at[idx_v.at[j0 + k]],
                sem,
                add=True,
            )
            for k in range(FIRE)
        ]
        for cp in cps:
            cp.wait()

    plsc.subcore_barrier()
    pltpu.sync_copy(acc_sh.at[pl.ds(sid * NPS, NPS)],
                    out_hbm.at[cid, pl.ds(sid * NPS, NPS)])


@jax.jit
def _sc_scatter(msg, dst3, zeros_nd):
    return pl.kernel(
        _scatter_body,
        out_type=jax.ShapeDtypeStruct((NC, N, D), jnp.float32),
        mesh=_sc_mesh(),
        compiler_params=pltpu.CompilerParams(use_tc_tiling_on_sc=False),
        scratch_types=[
            pltpu.VMEM((NCHUNK, WIN), jnp.int32),
            pltpu.VMEM((EPW, D), jnp.float32),
            pltpu.VMEM_SHARED((N, D), jnp.float32),
            pltpu.SemaphoreType.DMA,
        ],
    )(msg, dst3, zeros_nd)


def _stats_kernel(ea_ref, w1_ref, b1_ref, w2_ref, b2_ref,
                  s1_ref, ss1_ref, s2_ref, ss2_ref):
    i = pl.program_id(0)

    @pl.when(i == 0)
    def _():
        s1_ref[...] = jnp.zeros_like(s1_ref)
        ss1_ref[...] = jnp.zeros_like(ss1_ref)
        s2_ref[...] = jnp.zeros_like(s2_ref)
        ss2_ref[...] = jnp.zeros_like(ss2_ref)

    ea = ea_ref[...]
    r1 = jnp.maximum(_dot_bf16(ea, w1_ref[...]) + b1_ref[...], 0.0)
    r2 = jnp.maximum(_dot_bf16(ea, w2_ref[...]) + b2_ref[...], 0.0)
    s1_ref[...] += jnp.sum(r1, axis=0, keepdims=True)
    ss1_ref[...] += jnp.sum(r1 * r1, axis=0, keepdims=True)
    s2_ref[...] += jnp.sum(r2, axis=0, keepdims=True)
    ss2_ref[...] += jnp.sum(r2 * r2, axis=0, keepdims=True)


@jax.jit
def _tc_stats(edge_attr, w1, b1, w2, b2):
    one_row = pl.BlockSpec((1, DD), lambda i: (0, 0))
    return pl.pallas_call(
        _stats_kernel,
        grid=(NT,),
        in_specs=[
            pl.BlockSpec((ET, E_DIM), lambda i: (i, 0)),
            pl.BlockSpec((E_DIM, DD), lambda i: (0, 0)),
            pl.BlockSpec((1, DD), lambda i: (0, 0)),
            pl.BlockSpec((E_DIM, DD), lambda i: (0, 0)),
            pl.BlockSpec((1, DD), lambda i: (0, 0)),
        ],
        out_specs=[one_row, one_row, one_row, one_row],
        out_shape=[jax.ShapeDtypeStruct((1, DD), jnp.float32)] * 4,
    )(edge_attr, w1, b1, w2, b2)


def _msg_kernel(ea_ref, xj_ref, w_ref, b_ref, g_ref, bt_ref,
                s_ref, ss_ref, k1_ref, k2_ref, msg_ref):
    mu = s_ref[...] * (1.0 / NE)
    var = ss_ref[...] * (1.0 / NE) - mu * mu
    inv = jax.lax.rsqrt(var + EPS)
    sc = g_ref[...] * inv
    sh = bt_ref[...] - mu * sc

    ea = ea_ref[...]
    xj = xj_ref[...]
    a = _dot_bf16(ea, w_ref[...]) + b_ref[...]
    we = sc * jnp.maximum(a, 0.0) + sh
    xjb = _dot_bf16(xj, k1_ref[...])
    web = we.astype(jnp.bfloat16).astype(jnp.float32)
    p = xjb * web
    p_hi = p.astype(jnp.bfloat16).astype(jnp.float32)
    p_lo = p - p_hi
    msg_ref[...] = _dot_bf16(p_hi, k2_ref[...]) + _dot_bf16(p_lo, k2_ref[...])


@jax.jit
def _tc_msg(edge_attr, xj, w, b, g, bt, s, ss, k1, k2):
    row = pl.BlockSpec((1, DD), lambda i: (0, 0))
    return pl.pallas_call(
        _msg_kernel,
        grid=(NT,),
        in_specs=[
            pl.BlockSpec((ET, E_DIM), lambda i: (i, 0)),
            pl.BlockSpec((ET, D), lambda i: (i, 0)),
            pl.BlockSpec((E_DIM, DD), lambda i: (0, 0)),
            row, row, row, row, row,
            pl.BlockSpec((D, DD), lambda i: (0, 0)),
            pl.BlockSpec((DD, D), lambda i: (0, 0)),
        ],
        out_specs=pl.BlockSpec((ET, D), lambda i: (i, 0)),
        out_shape=jax.ShapeDtypeStruct((NE, D), jnp.float32),
    )(edge_attr, xj, w, b, g, bt, s, ss, k1, k2)


def _combine_kernel(p_ref, x_ref, root_ref, bias_ref, out_ref):
    out_ref[...] = (
        p_ref[0] + p_ref[1]
        + _dot_bf16(x_ref[...], root_ref[...])
        + bias_ref[...]
    )


@jax.jit
def _tc_combine(partials, x, root, bias_row):
    return pl.pallas_call(
        _combine_kernel,
        out_shape=jax.ShapeDtypeStruct((N, D), jnp.float32),
    )(partials, x, root, bias_row)


def _bn_rows(h, g, bt):
    mu = jnp.mean(h, axis=0, keepdims=True)
    dv = h - mu
    var = jnp.mean(dv * dv, axis=0, keepdims=True)
    return g * dv * jax.lax.rsqrt(var + EPS) + bt


def _head_kernel(x1_ref, p_ref, root2_ref, bias2_ref,
                 l1a_ref, l1b_ref, l1bias_ref, l1g_ref, l1bt_ref,
                 m1w_ref, m1b_ref, m1g_ref, m1bt_ref,
                 m2w_ref, m2b_ref, m2g_ref, m2bt_ref, out_ref):
    x1 = x1_ref[...]
    x2 = (p_ref[0] + p_ref[1]
          + _dot_bf16(x1, root2_ref[...])
          + bias2_ref[...])
    h = (_dot_bf16(x1, l1a_ref[...]) + _dot_bf16(x2, l1b_ref[...])
         + l1bias_ref[...])
    h = _bn_rows(jnp.maximum(h, 0.0), l1g_ref[...], l1bt_ref[...])
    h = _dot_bf16(h, m1w_ref[...]) + m1b_ref[...]
    h = _bn_rows(jnp.maximum(h, 0.0), m1g_ref[...], m1bt_ref[...])
    h = _dot_bf16(h, m2w_ref[...]) + m2b_ref[...]
    out_ref[...] = _bn_rows(jnp.maximum(h, 0.0), m2g_ref[...], m2bt_ref[...])


@jax.jit
def _tc_head(x1, partials2, root2, bias2_row, l1a, l1b, l1bias, l1g, l1bt,
             m1w, m1b, m1g, m1bt, m2w, m2b, m2g, m2bt):
    return pl.pallas_call(
        _head_kernel,
        out_shape=jax.ShapeDtypeStruct((N, C), jnp.float32),
    )(x1, partials2, root2, bias2_row, l1a, l1b, l1bias, l1g, l1bt,
      m1w, m1b, m1g, m1bt, m2w, m2b, m2g, m2bt)


def kernel(x, edge_index, edge_attr, nnW1, nnb1, nng1, nnbt1, root1, bias1,
           nnW2, nnb2, nng2, nnbt2, root2, bias2, lin1W, lin1b, lin1g, lin1bt,
           m1W, m1b, m1g, m1bt, m2W, m2b, m2g, m2bt):
    f32 = jnp.float32
    src3 = edge_index[0].astype(jnp.int32).reshape(NW, NCHUNK, WIN)
    dst3 = edge_index[1].astype(jnp.int32).reshape(NW, NCHUNK, WIN)
    zeros_nd = jnp.zeros((N, D), f32)

    b1r, g1r, bt1r = (v.reshape(1, DD) for v in (nnb1, nng1, nnbt1))
    b2r, g2r, bt2r = (v.reshape(1, DD) for v in (nnb2, nng2, nnbt2))
    bias1r = bias1.reshape(1, D)
    bias2r = bias2.reshape(1, D)
    l1biasr, l1gr, l1btr = (v.reshape(1, Q) for v in (lin1b, lin1g, lin1bt))
    m1br, m1gr, m1btr = (v.reshape(1, Q) for v in (m1b, m1g, m1bt))
    m2br, m2gr, m2btr = (v.reshape(1, C) for v in (m2b, m2g, m2bt))
    l1a = lin1W[:D]
    l1b = lin1W[D:]

    eye = jnp.eye(D, dtype=f32)
    k1 = jnp.kron(eye, jnp.ones((1, D), f32))
    k2 = jnp.kron(jnp.ones((D, 1), f32), eye)

    s1, ss1, s2, ss2 = _tc_stats(edge_attr, nnW1, b1r, nnW2, b2r)

    xj1 = _sc_gather(x, src3)
    msg1 = _tc_msg(edge_attr, xj1, nnW1, b1r, g1r, bt1r, s1, ss1, k1, k2)
    p1 = _sc_scatter(msg1, dst3, zeros_nd)
    x1 = _tc_combine(p1, x, root1, bias1r)

    xj2 = _sc_gather(x1, src3)
    msg2 = _tc_msg(edge_attr, xj2, nnW2, b2r, g2r, bt2r, s2, ss2, k1, k2)
    p2 = _sc_scatter(msg2, dst3, zeros_nd)

    return _tc_head(x1, p2, root2, bias2r, l1a, l1b, l1biasr, l1gr, l1btr,
                    m1W, m1br, m1gr, m1btr, m2W, m2br, m2gr, m2btr)
